# baseline (speedup 1.0000x reference)
import functools

import jax
import jax.numpy as jnp
from jax import lax
from jax.experimental import pallas as pl
from jax.experimental.pallas import tpu as pltpu
from jax.experimental.pallas import tpu_sc as plsc

BATCH = 4096
SEQ_LEN = 200
WORD_DIM = 64
VOCAB = 1000000

_NUM_CORES = 2
_NUM_SUBCORES = 16
_D_PER_CORE = WORD_DIM // _NUM_CORES
_L_MAX = 13
_HALF = BATCH
_U_MAX = _L_MAX


def _sc_kernel(ids_hbm, table_hbm, pos_hbm, out_hbm,
               i0, i1, i2, i3, g0, g1, g2, g3, w0, w1, w2, w3, t0,
               sbuf0):
    cid = lax.axis_index("c")
    sid = lax.axis_index("s")
    d_base = cid * _D_PER_CORE
    l0 = jnp.where(sid < 8, 13 * sid, 104 + 12 * (sid - 8))
    nl = jnp.where(sid < 8, 13, 12)
    nu = nl
    c0 = (l0 // 16) * 16

    def body(ix0, ix1, ix2, ix3, dst0, dst1, dst2, dst3, posall):
        idxs = (ix0, ix1, ix2, ix3)
        isems = (i0, i1, i2, i3)
        dsts = (dst0, dst1, dst2, dst3)
        gsems = (g0, g1, g2, g3)
        wsems = (w0, w1, w2, w3)

        def stage_idx(u, p):
            pltpu.async_copy(
                ids_hbm.at[l0 + u, pl.ds(0, _HALF)],
                idxs[p], isems[p])

        def wait_idx(p):
            pltpu.make_async_copy(
                ids_hbm.at[0, pl.ds(0, _HALF)], idxs[p], isems[p]).wait()

        def fire_gather(sbuf, p):
            wait_idx(p)
            pltpu.async_copy(sbuf.at[idxs[p]], dsts[p], gsems[p])

        def wait_gather(p):
            pltpu.make_async_copy(
                table_hbm.at[0, pl.ds(0, _HALF)], dsts[p], gsems[p]).wait()

        def fire_write(u, d_glob, p):
            pltpu.async_copy(
                dsts[p],
                out_hbm.at[l0 + u, d_glob, pl.ds(0, _HALF)],
                wsems[p])

        def wait_write(p):
            pltpu.make_async_copy(
                dsts[p], out_hbm.at[0, 0, pl.ds(0, _HALF)], wsems[p]).wait()

        def add_unit(p, d_loc, u):
            dst = dsts[p]
            po = l0 + u - c0
            p16 = posall[d_loc, pl.ds((po // 16) * 16, 16)]
            splat = p16.at[jnp.full((16,), po % 16, jnp.int32)].get(
                mode="promise_in_bounds")

            def blk(k, carry):
                for j in range(8):
                    dst[pl.ds(k * 128 + j * 16, 16)] += splat
                return carry
            lax.fori_loop(0, _HALF // 128, blk, 0)

        pltpu.sync_copy(pos_hbm.at[pl.ds(d_base, _D_PER_CORE), pl.ds(c0, 32)],
                        posall)
        for p in range(4):
            stage_idx(p, p)

        @pl.when(sid == 0)
        def _():
            pltpu.async_copy(table_hbm.at[d_base + 0], sbuf0, t0)

        def d_body(d_loc, carry):
                sbuf, tsem = sbuf0, t0
                d_glob = d_base + d_loc
                not_first = d_loc > 0

                @pl.when(sid == 0)
                def _():
                    pltpu.make_async_copy(table_hbm.at[0], sbuf, tsem).wait()
                plsc.subcore_barrier()

                for q in (0, 1):
                    @pl.when(not_first)
                    def _():
                        wait_write(q)
                    fire_gather(sbuf, q)

                def quad(j, carry):
                    for q in range(4):
                        u = 4 * j + q

                        @pl.when(u < nu)
                        def _():
                            wait_gather(q)
                            @pl.when(u + 4 < nu)
                            def _():
                                stage_idx(u + 4, q)
                            nxt = u + 2
                            pn = (q + 2) % 4
                            if q < 2:
                                drain_ok = jnp.logical_or(j > 0, not_first)
                            else:
                                drain_ok = jnp.full((), True)

                            @pl.when(nxt < nu)
                            def _():
                                @pl.when(drain_ok)
                                def _():
                                    wait_write(pn)
                                fire_gather(sbuf, pn)
                            add_unit(q, d_loc, u)
                            fire_write(u, d_glob, q)
                    return carry
                lax.fori_loop(0, (_U_MAX + 3) // 4, quad, 0)

                plsc.subcore_barrier()

                @pl.when(jnp.logical_and(sid == 0, d_loc + 1 < _D_PER_CORE))
                def _():
                    pltpu.async_copy(table_hbm.at[d_base + d_loc + 1],
                                     sbuf, tsem)

                @pl.when(d_loc + 1 < _D_PER_CORE)
                def _():
                    for p in range(4):
                        stage_idx(p, p)
                return carry

        lax.fori_loop(0, _D_PER_CORE, d_body, 0)

        for p in range(4):
            wait_write(p)

    pl.run_scoped(
        body,
        pltpu.VMEM((_HALF,), jnp.int32),
        pltpu.VMEM((_HALF,), jnp.int32),
        pltpu.VMEM((_HALF,), jnp.int32),
        pltpu.VMEM((_HALF,), jnp.int32),
        pltpu.VMEM((_HALF,), jnp.float32),
        pltpu.VMEM((_HALF,), jnp.float32),
        pltpu.VMEM((_HALF,), jnp.float32),
        pltpu.VMEM((_HALF,), jnp.float32),
        pltpu.VMEM((_D_PER_CORE, 32), jnp.float32),
    )


@jax.jit
def _run(ids_t, table_t, pos_t):
    mesh = plsc.VectorSubcoreMesh(core_axis_name="c", subcore_axis_name="s")
    f = functools.partial(
        pl.kernel,
        mesh=mesh,
        out_type=jax.ShapeDtypeStruct((SEQ_LEN, WORD_DIM, BATCH), jnp.float32),
        scratch_types=(
            [pltpu.SemaphoreType.DMA] * 13
            + [pltpu.VMEM_SHARED((VOCAB,), jnp.float32)]
        ),
        compiler_params=pltpu.CompilerParams(use_tc_tiling_on_sc=False),
    )(_sc_kernel)
    return f(ids_t, table_t, pos_t)


def kernel(input_ids, word_table, pos_table):
    ids_t = input_ids.T
    table_t = word_table.T
    pos_t = pos_table.T
    out_t = _run(ids_t, table_t, pos_t)
    return jnp.transpose(out_t, (2, 0, 1))


# speedup vs baseline: 4.7491x; 4.7491x over previous
import functools

import jax
import jax.numpy as jnp
from jax import lax
from jax.experimental import pallas as pl
from jax.experimental.pallas import tpu as pltpu
from jax.experimental.pallas import tpu_sc as plsc

BATCH = 4096
SEQ_LEN = 200
WORD_DIM = 64

_NUM_CORES = 2
_NUM_SUBCORES = 16
_NUM_WORKERS = _NUM_CORES * _NUM_SUBCORES
_ROWS_PER_WORKER = BATCH // _NUM_WORKERS
_IDS_PER_WORKER = _ROWS_PER_WORKER * SEQ_LEN

_CHUNK0 = 128
_CHUNK1 = SEQ_LEN - _CHUNK0

_NBUF = 4
_QUADS = _ROWS_PER_WORKER // _NBUF


def _sc_kernel(ids_hbm, table_hbm, pos_hbm, out_hbm,
               idxall, pos_v, out_bufs, g_sems, st_sems):
    cid = lax.axis_index("c")
    sid = lax.axis_index("s")
    wid = sid * _NUM_CORES + cid
    flat_base = wid * _IDS_PER_WORKER

    pltpu.sync_copy(pos_hbm.at[pl.ds(0, SEQ_LEN)], pos_v)
    pltpu.sync_copy(ids_hbm.at[pl.ds(flat_base, _IDS_PER_WORKER)], idxall)

    def fire_gather(r_loc, j):
        o = r_loc * SEQ_LEN
        pltpu.async_copy(table_hbm.at[idxall.at[pl.ds(o, _CHUNK0)]],
                         out_bufs[j].at[pl.ds(0, _CHUNK0)], g_sems[j])
        pltpu.async_copy(table_hbm.at[idxall.at[pl.ds(o + _CHUNK0, _CHUNK1)]],
                         out_bufs[j].at[pl.ds(_CHUNK0, _CHUNK1)], g_sems[j])

    def wait_gather(j):
        pltpu.make_async_copy(table_hbm.at[pl.ds(0, SEQ_LEN)],
                              out_bufs[j], g_sems[j]).wait()

    def fire_store(r_loc, j):
        pltpu.async_copy(out_bufs[j],
                         out_hbm.at[pl.ds(flat_base + r_loc * SEQ_LEN, SEQ_LEN)],
                         st_sems[j])

    def wait_store(j):
        pltpu.make_async_copy(out_bufs[j],
                              out_hbm.at[pl.ds(0, SEQ_LEN)], st_sems[j]).wait()

    def add_pos(j):
        def body8(r8, c):
            r = r8 * 8
            for rr in range(8):
                for k in range(WORD_DIM // 16):
                    plsc.addupdate(out_bufs[j].at[r + rr, pl.ds(k * 16, 16)],
                                   pos_v[r + rr, pl.ds(k * 16, 16)])
            return c
        lax.fori_loop(0, SEQ_LEN // 8, body8, 0)

    for j in range(_NBUF):
        fire_gather(j, j)

    def quad_body(i, carry):
        for j in range(_NBUF):
            r = i * _NBUF + j
            wait_gather(j)
            add_pos(j)
            fire_store(r, j)
            if j >= 1:
                @pl.when(i < _QUADS - 1)
                def _():
                    wait_store(j - 1)
                    fire_gather(i * _NBUF + _NBUF + (j - 1), j - 1)

        @pl.when(i < _QUADS - 1)
        def _():
            wait_store(_NBUF - 1)
            fire_gather(i * _NBUF + _NBUF + (_NBUF - 1), _NBUF - 1)
        return carry

    lax.fori_loop(0, _QUADS, quad_body, 0)

    for j in range(_NBUF):
        wait_store(j)


def _wrapped(ids_hbm, table_hbm, pos_hbm, out_hbm,
             idxall, pos_v, b0, b1, b2, b3,
             g0, g1, g2, g3, s0, s1, s2, s3):
    _sc_kernel(ids_hbm, table_hbm, pos_hbm, out_hbm, idxall, pos_v,
               [b0, b1, b2, b3], [g0, g1, g2, g3], [s0, s1, s2, s3])


@jax.jit
def _run(ids_flat, word_table, pos_table):
    mesh = plsc.VectorSubcoreMesh(core_axis_name="c", subcore_axis_name="s")
    f = functools.partial(
        pl.kernel,
        mesh=mesh,
        out_type=jax.ShapeDtypeStruct((BATCH * SEQ_LEN, WORD_DIM), jnp.float32),
        scratch_types=(
            [pltpu.VMEM((_IDS_PER_WORKER,), jnp.int32),
             pltpu.VMEM((SEQ_LEN, WORD_DIM), jnp.float32)]
            + [pltpu.VMEM((SEQ_LEN, WORD_DIM), jnp.float32)] * _NBUF
            + [pltpu.SemaphoreType.DMA] * (2 * _NBUF)
        ),
        compiler_params=pltpu.CompilerParams(use_tc_tiling_on_sc=False),
    )(_wrapped)
    return f(ids_flat, word_table, pos_table)


def kernel(input_ids, word_table, pos_table):
    ids_flat = input_ids.reshape(-1).astype(jnp.int32)
    out = _run(ids_flat, word_table, pos_table)
    return out.reshape(BATCH, SEQ_LEN, WORD_DIM)
